# baseline (speedup 1.0000x reference)
import functools

import jax
import jax.numpy as jnp
from jax import lax
from jax.experimental import pallas as pl
from jax.experimental.pallas import tpu as pltpu
from jax.experimental.pallas import tpu_sc as plsc

N = 10000
E = 320000
D = 128
L = 3

NC = 2
NS = 16
NQ = 2
QF = D // (NC * NQ)

NP = 10240
BR = 2000
NB = N // BR
EPT = E // NS
RPT = NP // NS
ZR = 128


CB = 128
NFULL = EPT // CB
NBUF = 4
NQUAD = NFULL // NBUF
TAIL = EPT - NFULL * CB


def _sc_agg_call(h4, src2, dst2):
    mesh = plsc.VectorSubcoreMesh(
        core_axis_name="c", subcore_axis_name="s", num_cores=NC, num_subcores=NS
    )

    @functools.partial(
        pl.kernel,
        mesh=mesh,
        compiler_params=pltpu.CompilerParams(use_tc_tiling_on_sc=False),
        out_type=jax.ShapeDtypeStruct((NC, NQ, NP, QF), jnp.float32),
        scratch_types=[
            pltpu.VMEM((EPT,), jnp.int32),
            pltpu.VMEM((EPT,), jnp.int32),
            [pltpu.VMEM((CB,), jnp.int32) for _ in range(NBUF)],
            [pltpu.VMEM((CB,), jnp.int32) for _ in range(NBUF)],
            [pltpu.VMEM((CB, QF), jnp.float32) for _ in range(NBUF)],
            pltpu.VMEM((TAIL,), jnp.int32),
            pltpu.VMEM((TAIL,), jnp.int32),
            pltpu.VMEM((TAIL, QF), jnp.float32),
            pltpu.VMEM((RPT, QF), jnp.float32),
            pltpu.VMEM_SHARED((NP, QF), jnp.float32),
            [pltpu.SemaphoreType.DMA for _ in range(NBUF)],
            [pltpu.SemaphoreType.DMA for _ in range(NBUF)],
        ],
    )
    def k(h_hbm, src_hbm, dst_hbm, out_hbm, src_v, dst_v, srcb, dstb, rows,
          srct_v, dstt_v, rows_t, bounce, agg_sh, gsem, ssem):
        cid = lax.axis_index("c")
        sid = lax.axis_index("s")

        pltpu.sync_copy(src_hbm.at[sid], src_v)
        pltpu.sync_copy(dst_hbm.at[sid], dst_v)

        def zstore(i, _):
            r = i // (QF // 16)
            col = (i % (QF // 16)) * 16
            bounce[r, pl.ds(col, 16)] = jnp.zeros((16,), jnp.float32)
            return 0

        lax.fori_loop(0, RPT * (QF // 16), zstore, 0)

        for q in range(NQ):
            pltpu.sync_copy(bounce, agg_sh.at[pl.ds(sid * RPT, RPT)])

            plsc.subcore_barrier()

            base = (cid * NQ + q) * NP

            def prep_src(buf, j):
                for kk in range(CB // 16):
                    buf[pl.ds(kk * 16, 16)] = (
                        src_v[pl.ds(j * CB + kk * 16, 16)] + base)

            def prep_dst(buf, j):
                for kk in range(CB // 16):
                    buf[pl.ds(kk * 16, 16)] = dst_v[pl.ds(j * CB + kk * 16, 16)]

            for b in range(NBUF):
                prep_src(srcb[b], b)
                pltpu.async_copy(h_hbm.at[srcb[b]], rows[b], gsem[b])

            def quad(i, _):
                j0 = i * NBUF
                for b in range(NBUF):
                    pltpu.make_async_copy(
                        h_hbm.at[srcb[b]], rows[b], gsem[b]).wait()
                    prep_dst(dstb[b], j0 + b)
                    pltpu.async_copy(
                        rows[b], agg_sh.at[dstb[b]], ssem[b], add=True)
                for b in range(NBUF):
                    @pl.when(j0 + b + NBUF < NFULL)
                    def _():
                        pltpu.make_async_copy(
                            rows[b], agg_sh.at[dstb[b]], ssem[b]).wait()
                        prep_src(srcb[b], j0 + b + NBUF)
                        pltpu.async_copy(h_hbm.at[srcb[b]], rows[b], gsem[b])
                return 0

            lax.fori_loop(0, NQUAD, quad, 0)

            for b in range(NBUF):
                pltpu.make_async_copy(
                    rows[b], agg_sh.at[dstb[b]], ssem[b]).wait()

            for kk in range(TAIL // 16):
                srct_v[pl.ds(kk * 16, 16)] = (
                    src_v[pl.ds(NFULL * CB + kk * 16, 16)] + base)
                dstt_v[pl.ds(kk * 16, 16)] = (
                    dst_v[pl.ds(NFULL * CB + kk * 16, 16)])
            pltpu.async_copy(h_hbm.at[srct_v], rows_t, gsem[0]).wait()
            pltpu.sync_copy(rows_t, agg_sh.at[dstt_v], add=True)

            plsc.subcore_barrier()

            pltpu.sync_copy(agg_sh.at[pl.ds(sid * RPT, RPT)], bounce)
            pltpu.sync_copy(bounce, out_hbm.at[cid, q, pl.ds(sid * RPT, RPT)])

            if q + 1 < NQ:
                lax.fori_loop(0, RPT * (QF // 16), zstore, 0)
                plsc.subcore_barrier()

    return k(h4.reshape(NC * NQ * NP, QF), src2, dst2)


def _lin0_body(x_ref, w_ref, b_ref, out_ref):
    h = jnp.dot(x_ref[...], w_ref[...], preferred_element_type=jnp.float32) + b_ref[...]
    for c in range(NC):
        for q in range(NQ):
            out_ref[c, q] = h[:, (c * NQ + q) * QF:(c * NQ + q + 1) * QF]


def _lin0_call(x, W0, b0):
    full = lambda shape: pl.BlockSpec(shape, lambda i: tuple(0 for _ in shape))
    return pl.pallas_call(
        _lin0_body,
        grid=(NB,),
        in_specs=[pl.BlockSpec((BR, D), lambda i: (i, 0)), full((D, D)),
                  full((1, D))],
        out_specs=pl.BlockSpec((NC, NQ, BR, QF), lambda i: (0, 0, i, 0)),
        out_shape=jax.ShapeDtypeStruct((NC, NQ, NP, QF), jnp.float32),
    )(x, W0, b0)


def _mlp_body(h_ref, a_ref, w1_ref, b1_ref, w2_ref, b2_ref, g_ref, be_ref,
              out_ref, split_ref, z_ref, acc_ref):
    p = pl.program_id(0)
    i = pl.program_id(1)

    @pl.when(p == 0)
    def _():
        z = jnp.concatenate(
            [h_ref[c, q] + a_ref[c, q] for c in range(NC) for q in range(NQ)],
            axis=1)
        z = jnp.maximum(
            jnp.dot(z, w1_ref[...], preferred_element_type=jnp.float32)
            + b1_ref[...], 0.0)
        z = jnp.maximum(
            jnp.dot(z, w2_ref[...], preferred_element_type=jnp.float32)
            + b2_ref[...], 0.0)
        z_ref[pl.ds(i * BR, BR), :] = z
        s = jnp.sum(z, axis=0, keepdims=True)
        sq = jnp.sum(z * z, axis=0, keepdims=True)

        @pl.when(i == 0)
        def _():
            acc_ref[0:1] = s
            acc_ref[1:2] = sq

        @pl.when(i > 0)
        def _():
            acc_ref[0:1] += s
            acc_ref[1:2] += sq

    @pl.when(p == 1)
    def _():
        mean = acc_ref[0:1] / N
        var = acc_ref[1:2] / N - mean * mean
        z = z_ref[pl.ds(i * BR, BR), :]
        t = jnp.tanh((z - mean) * lax.rsqrt(var + 1e-5) * g_ref[...]
                     + be_ref[...])
        out_ref[...] = t
        for c in range(NC):
            for q in range(NQ):
                split_ref[c, q] = t[:, (c * NQ + q) * QF:(c * NQ + q + 1) * QF]


def _mlp_call(h4, agg4, W1l, B1l, W2l, B2l, Gl, Bel):
    blk4_in = pl.BlockSpec(
        (NC, NQ, BR, QF), lambda p, i: (0, 0, jnp.where(p == 0, i, 0), 0))
    full = lambda shape: pl.BlockSpec(shape, lambda p, i: tuple(0 for _ in shape))
    return pl.pallas_call(
        _mlp_body,
        grid=(2, NB),
        in_specs=[blk4_in, blk4_in, full((D, D)), full((1, D)), full((D, D)),
                  full((1, D)), full((1, D)), full((1, D))],
        out_specs=(
            pl.BlockSpec((BR, D), lambda p, i: (jnp.where(p == 1, i, 0), 0)),
            pl.BlockSpec((NC, NQ, BR, QF),
                         lambda p, i: (0, 0, jnp.where(p == 1, i, 0), 0)),
        ),
        out_shape=(jax.ShapeDtypeStruct((N, D), jnp.float32),
                   jax.ShapeDtypeStruct((NC, NQ, NP, QF), jnp.float32)),
        scratch_shapes=[pltpu.VMEM((N, D), jnp.float32),
                        pltpu.VMEM((8, D), jnp.float32)],
    )(h4, agg4, W1l, B1l, W2l, B2l, Gl, Bel)


def kernel(x, edge_index, W0, b0, W1, B1, W2, B2, G, Be):
    src2 = edge_index[0].reshape(NS, EPT)
    dst2 = edge_index[1].reshape(NS, EPT)

    h4 = _lin0_call(x, W0, b0.reshape(1, D))

    ws = (W1, B1.reshape(L, 1, D), W2, B2.reshape(L, 1, D),
          G.reshape(L, 1, D), Be.reshape(L, 1, D))

    def step(h4c, w):
        w1, b1, w2, b2, g, be = w
        agg4 = _sc_agg_call(h4c, src2, dst2)
        h_full, h4n = _mlp_call(h4c, agg4, w1, b1, w2, b2, g, be)
        return h4n, h_full

    _, hs = lax.scan(step, h4, ws)
    return (x, hs[0], hs[1], hs[2])


# speedup vs baseline: 1.0569x; 1.0569x over previous
import functools

import jax
import jax.numpy as jnp
from jax import lax
from jax.experimental import pallas as pl
from jax.experimental.pallas import tpu as pltpu
from jax.experimental.pallas import tpu_sc as plsc

N = 10000
E = 320000
D = 128
L = 3

NC = 2
NS = 16
NQ = 2
QF = D // (NC * NQ)

NP = 10240
BR = 2000
NB = N // BR
EPT = E // NS
RPT = NP // NS
ZR = 128


CB = 128
NFULL = EPT // CB
NBUF = 4
NQUAD = NFULL // NBUF
TAIL = EPT - NFULL * CB


def _sc_agg_call(h4, src2, dst2):
    mesh = plsc.VectorSubcoreMesh(
        core_axis_name="c", subcore_axis_name="s", num_cores=NC, num_subcores=NS
    )

    @functools.partial(
        pl.kernel,
        mesh=mesh,
        compiler_params=pltpu.CompilerParams(use_tc_tiling_on_sc=False),
        out_type=jax.ShapeDtypeStruct((NC, NQ, NP, QF), jnp.float32),
        scratch_types=[
            pltpu.VMEM((EPT,), jnp.int32),
            pltpu.VMEM((EPT,), jnp.int32),
            [pltpu.VMEM((CB,), jnp.int32) for _ in range(NBUF)],
            [pltpu.VMEM((CB,), jnp.int32) for _ in range(NBUF)],
            [pltpu.VMEM((CB, QF), jnp.float32) for _ in range(NBUF)],
            pltpu.VMEM((TAIL,), jnp.int32),
            pltpu.VMEM((TAIL,), jnp.int32),
            pltpu.VMEM((TAIL, QF), jnp.float32),
            pltpu.VMEM((RPT, QF), jnp.float32),
            pltpu.VMEM_SHARED((NP, QF), jnp.float32),
            [pltpu.SemaphoreType.DMA for _ in range(NBUF)],
            [pltpu.SemaphoreType.DMA for _ in range(NBUF)],
        ],
    )
    def k(h_hbm, src_hbm, dst_hbm, out_hbm, src_v, dst_v, srcb, dstb, rows,
          srct_v, dstt_v, rows_t, bounce, agg_sh, gsem, ssem):
        cid = lax.axis_index("c")
        sid = lax.axis_index("s")

        pltpu.sync_copy(src_hbm.at[sid], src_v)
        pltpu.sync_copy(dst_hbm.at[sid], dst_v)

        def zstore(i, _):
            r = i // (QF // 16)
            col = (i % (QF // 16)) * 16
            bounce[r, pl.ds(col, 16)] = jnp.zeros((16,), jnp.float32)
            return 0

        lax.fori_loop(0, RPT * (QF // 16), zstore, 0)

        for q in range(NQ):
            pltpu.sync_copy(bounce, agg_sh.at[pl.ds(sid * RPT, RPT)])

            plsc.subcore_barrier()

            base = (cid * NQ + q) * NP

            def prep_src(buf, j):
                for kk in range(CB // 16):
                    buf[pl.ds(kk * 16, 16)] = (
                        src_v[pl.ds(j * CB + kk * 16, 16)] + base)

            def prep_dst(buf, j):
                for kk in range(CB // 16):
                    buf[pl.ds(kk * 16, 16)] = dst_v[pl.ds(j * CB + kk * 16, 16)]

            for b in range(NBUF):
                prep_src(srcb[b], b)
                pltpu.async_copy(h_hbm.at[srcb[b]], rows[b], gsem[b])

            def quad(i, _):
                j0 = i * NBUF
                for b in range(NBUF):
                    pltpu.make_async_copy(
                        h_hbm.at[srcb[b]], rows[b], gsem[b]).wait()
                    prep_dst(dstb[b], j0 + b)
                    pltpu.async_copy(
                        rows[b], agg_sh.at[dstb[b]], ssem[b], add=True)
                for b in range(NBUF):
                    @pl.when(j0 + b + NBUF < NFULL)
                    def _():
                        pltpu.make_async_copy(
                            rows[b], agg_sh.at[dstb[b]], ssem[b]).wait()
                        prep_src(srcb[b], j0 + b + NBUF)
                        pltpu.async_copy(h_hbm.at[srcb[b]], rows[b], gsem[b])
                return 0

            lax.fori_loop(0, NQUAD, quad, 0)

            for b in range(NBUF):
                pltpu.make_async_copy(
                    rows[b], agg_sh.at[dstb[b]], ssem[b]).wait()

            for kk in range(TAIL // 16):
                srct_v[pl.ds(kk * 16, 16)] = (
                    src_v[pl.ds(NFULL * CB + kk * 16, 16)] + base)
                dstt_v[pl.ds(kk * 16, 16)] = (
                    dst_v[pl.ds(NFULL * CB + kk * 16, 16)])
            pltpu.async_copy(h_hbm.at[srct_v], rows_t, gsem[0]).wait()
            pltpu.sync_copy(rows_t, agg_sh.at[dstt_v], add=True)

            plsc.subcore_barrier()

            pltpu.sync_copy(agg_sh.at[pl.ds(sid * RPT, RPT)], bounce)
            pltpu.sync_copy(bounce, out_hbm.at[cid, q, pl.ds(sid * RPT, RPT)])

            if q + 1 < NQ:
                lax.fori_loop(0, RPT * (QF // 16), zstore, 0)
                plsc.subcore_barrier()

    return k(h4.reshape(NC * NQ * NP, QF), src2, dst2)


def _lin0_body(x_ref, w_ref, b_ref, out_ref):
    h = jnp.dot(x_ref[...], w_ref[...], preferred_element_type=jnp.float32) + b_ref[...]
    for c in range(NC):
        for q in range(NQ):
            out_ref[c, q] = h[:, (c * NQ + q) * QF:(c * NQ + q + 1) * QF]


def _lin0_call(x, W0, b0):
    full = lambda shape: pl.BlockSpec(shape, lambda i: tuple(0 for _ in shape))
    return pl.pallas_call(
        _lin0_body,
        grid=(NB,),
        in_specs=[pl.BlockSpec((BR, D), lambda i: (i, 0)), full((D, D)),
                  full((1, D))],
        out_specs=pl.BlockSpec((NC, NQ, BR, QF), lambda i: (0, 0, i, 0)),
        out_shape=jax.ShapeDtypeStruct((NC, NQ, NP, QF), jnp.float32),
    )(x, W0, b0)


def _mlp1_body(h_ref, a_ref, w1_ref, b1_ref, w2_ref, b2_ref,
               z_ref, sum_ref, sq_ref, acc_ref):
    i = pl.program_id(0)
    z = jnp.concatenate(
        [h_ref[c, q] + a_ref[c, q] for c in range(NC) for q in range(NQ)],
        axis=1)
    z = jnp.maximum(
        jnp.dot(z, w1_ref[...], preferred_element_type=jnp.float32)
        + b1_ref[...], 0.0)
    z = jnp.maximum(
        jnp.dot(z, w2_ref[...], preferred_element_type=jnp.float32)
        + b2_ref[...], 0.0)
    z_ref[...] = z
    s = jnp.sum(z, axis=0, keepdims=True)
    sq = jnp.sum(z * z, axis=0, keepdims=True)

    @pl.when(i == 0)
    def _():
        acc_ref[0:1] = s
        acc_ref[1:2] = sq

    @pl.when(i > 0)
    def _():
        acc_ref[0:1] += s
        acc_ref[1:2] += sq

    sum_ref[...] = acc_ref[0:1]
    sq_ref[...] = acc_ref[1:2]


def _mlp2_body(z_ref, sum_ref, sq_ref, g_ref, be_ref, out_ref, split_ref):
    mean = sum_ref[...] / N
    var = sq_ref[...] / N - mean * mean
    t = jnp.tanh((z_ref[...] - mean) * lax.rsqrt(var + 1e-5) * g_ref[...]
                 + be_ref[...])
    out_ref[...] = t
    for c in range(NC):
        for q in range(NQ):
            split_ref[c, q] = t[:, (c * NQ + q) * QF:(c * NQ + q + 1) * QF]


def _mlp_call(h4, agg4, W1l, B1l, W2l, B2l, Gl, Bel):
    blk4 = pl.BlockSpec((NC, NQ, BR, QF), lambda i: (0, 0, i, 0))
    blkz = pl.BlockSpec((BR, D), lambda i: (i, 0))
    full = lambda shape: pl.BlockSpec(shape, lambda i: tuple(0 for _ in shape))
    z, s, sq = pl.pallas_call(
        _mlp1_body,
        grid=(NB,),
        in_specs=[blk4, blk4, full((D, D)), full((1, D)), full((D, D)),
                  full((1, D))],
        out_specs=(blkz, full((1, D)), full((1, D))),
        out_shape=(jax.ShapeDtypeStruct((N, D), jnp.float32),
                   jax.ShapeDtypeStruct((1, D), jnp.float32),
                   jax.ShapeDtypeStruct((1, D), jnp.float32)),
        scratch_shapes=[pltpu.VMEM((8, D), jnp.float32)],
    )(h4, agg4, W1l, B1l, W2l, B2l)
    return pl.pallas_call(
        _mlp2_body,
        grid=(NB,),
        in_specs=[blkz, full((1, D)), full((1, D)), full((1, D)),
                  full((1, D))],
        out_specs=(blkz, pl.BlockSpec((NC, NQ, BR, QF),
                                      lambda i: (0, 0, i, 0))),
        out_shape=(jax.ShapeDtypeStruct((N, D), jnp.float32),
                   jax.ShapeDtypeStruct((NC, NQ, NP, QF), jnp.float32)),
    )(z, s, sq, Gl, Bel)


def kernel(x, edge_index, W0, b0, W1, B1, W2, B2, G, Be):
    src2 = edge_index[0].reshape(NS, EPT)
    dst2 = edge_index[1].reshape(NS, EPT)

    h4 = _lin0_call(x, W0, b0.reshape(1, D))

    ws = (W1, B1.reshape(L, 1, D), W2, B2.reshape(L, 1, D),
          G.reshape(L, 1, D), Be.reshape(L, 1, D))

    def step(h4c, w):
        w1, b1, w2, b2, g, be = w
        agg4 = _sc_agg_call(h4c, src2, dst2)
        h_full, h4n = _mlp_call(h4c, agg4, w1, b1, w2, b2, g, be)
        return h4n, h_full

    _, hs = lax.scan(step, h4, ws)
    return (x, hs[0], hs[1], hs[2])


# speedup vs baseline: 1.1207x; 1.0603x over previous
import functools

import jax
import jax.numpy as jnp
from jax import lax
from jax.experimental import pallas as pl
from jax.experimental.pallas import tpu as pltpu
from jax.experimental.pallas import tpu_sc as plsc

N = 10000
E = 320000
D = 128
L = 3

NC = 2
NS = 16
NQ = 2
QF = D // (NC * NQ)

NP = 10240
BR = 2000
NB = N // BR
EPT = E // NS
RPT = NP // NS
ZR = 128


CB = 128
NFULL = EPT // CB
NBUF = 6
NQUAD = NFULL // NBUF
TAIL = EPT - NFULL * CB


def _sc_agg_call(h4, src2, dst2):
    mesh = plsc.VectorSubcoreMesh(
        core_axis_name="c", subcore_axis_name="s", num_cores=NC, num_subcores=NS
    )

    @functools.partial(
        pl.kernel,
        mesh=mesh,
        compiler_params=pltpu.CompilerParams(use_tc_tiling_on_sc=False),
        out_type=jax.ShapeDtypeStruct((NC, NQ, NP, QF), jnp.float32),
        scratch_types=[
            pltpu.VMEM((EPT,), jnp.int32),
            pltpu.VMEM((EPT,), jnp.int32),
            [pltpu.VMEM((CB,), jnp.int32) for _ in range(NBUF)],
            [pltpu.VMEM((CB,), jnp.int32) for _ in range(NBUF)],
            [pltpu.VMEM((CB, QF), jnp.float32) for _ in range(NBUF)],
            pltpu.VMEM((TAIL,), jnp.int32),
            pltpu.VMEM((TAIL,), jnp.int32),
            pltpu.VMEM((TAIL, QF), jnp.float32),
            pltpu.VMEM((RPT, QF), jnp.float32),
            pltpu.VMEM_SHARED((NP, QF), jnp.float32),
            [pltpu.SemaphoreType.DMA for _ in range(NBUF)],
            [pltpu.SemaphoreType.DMA for _ in range(NBUF)],
        ],
    )
    def k(h_hbm, src_hbm, dst_hbm, out_hbm, src_v, dst_v, srcb, dstb, rows,
          srct_v, dstt_v, rows_t, bounce, agg_sh, gsem, ssem):
        cid = lax.axis_index("c")
        sid = lax.axis_index("s")

        pltpu.sync_copy(src_hbm.at[sid], src_v)
        pltpu.sync_copy(dst_hbm.at[sid], dst_v)

        def zstore(i, _):
            r = i // (QF // 16)
            col = (i % (QF // 16)) * 16
            bounce[r, pl.ds(col, 16)] = jnp.zeros((16,), jnp.float32)
            return 0

        lax.fori_loop(0, RPT * (QF // 16), zstore, 0)

        for q in range(NQ):
            pltpu.sync_copy(bounce, agg_sh.at[pl.ds(sid * RPT, RPT)])

            plsc.subcore_barrier()

            base = (cid * NQ + q) * NP

            def prep_src(buf, j):
                for kk in range(CB // 16):
                    buf[pl.ds(kk * 16, 16)] = (
                        src_v[pl.ds(j * CB + kk * 16, 16)] + base)

            def prep_dst(buf, j):
                for kk in range(CB // 16):
                    buf[pl.ds(kk * 16, 16)] = dst_v[pl.ds(j * CB + kk * 16, 16)]

            for b in range(NBUF):
                prep_src(srcb[b], b)
                pltpu.async_copy(h_hbm.at[srcb[b]], rows[b], gsem[b])

            def quad(i, _):
                j0 = i * NBUF
                for b in range(NBUF):
                    pltpu.make_async_copy(
                        h_hbm.at[srcb[b]], rows[b], gsem[b]).wait()
                    prep_dst(dstb[b], j0 + b)
                    pltpu.async_copy(
                        rows[b], agg_sh.at[dstb[b]], ssem[b], add=True)
                for b in range(NBUF):
                    @pl.when(j0 + b + NBUF < NFULL)
                    def _():
                        pltpu.make_async_copy(
                            rows[b], agg_sh.at[dstb[b]], ssem[b]).wait()
                        prep_src(srcb[b], j0 + b + NBUF)
                        pltpu.async_copy(h_hbm.at[srcb[b]], rows[b], gsem[b])
                return 0

            lax.fori_loop(0, NQUAD, quad, 0)

            for b in range(NBUF):
                pltpu.make_async_copy(
                    rows[b], agg_sh.at[dstb[b]], ssem[b]).wait()

            for kk in range(TAIL // 16):
                srct_v[pl.ds(kk * 16, 16)] = (
                    src_v[pl.ds(NFULL * CB + kk * 16, 16)] + base)
                dstt_v[pl.ds(kk * 16, 16)] = (
                    dst_v[pl.ds(NFULL * CB + kk * 16, 16)])
            pltpu.async_copy(h_hbm.at[srct_v], rows_t, gsem[0]).wait()
            pltpu.sync_copy(rows_t, agg_sh.at[dstt_v], add=True)

            plsc.subcore_barrier()

            pltpu.sync_copy(agg_sh.at[pl.ds(sid * RPT, RPT)], bounce)
            pltpu.sync_copy(bounce, out_hbm.at[cid, q, pl.ds(sid * RPT, RPT)])

            if q + 1 < NQ:
                lax.fori_loop(0, RPT * (QF // 16), zstore, 0)
                plsc.subcore_barrier()

    return k(h4.reshape(NC * NQ * NP, QF), src2, dst2)


def _lin0_body(x_ref, w_ref, b_ref, out_ref):
    h = jnp.dot(x_ref[...], w_ref[...], preferred_element_type=jnp.float32) + b_ref[...]
    for c in range(NC):
        for q in range(NQ):
            out_ref[c, q] = h[:, (c * NQ + q) * QF:(c * NQ + q + 1) * QF]


def _lin0_call(x, W0, b0):
    full = lambda shape: pl.BlockSpec(shape, lambda i: tuple(0 for _ in shape))
    return pl.pallas_call(
        _lin0_body,
        grid=(NB,),
        in_specs=[pl.BlockSpec((BR, D), lambda i: (i, 0)), full((D, D)),
                  full((1, D))],
        out_specs=pl.BlockSpec((NC, NQ, BR, QF), lambda i: (0, 0, i, 0)),
        out_shape=jax.ShapeDtypeStruct((NC, NQ, NP, QF), jnp.float32),
    )(x, W0, b0)


def _mlp1_body(h_ref, a_ref, w1_ref, b1_ref, w2_ref, b2_ref,
               z_ref, sum_ref, sq_ref, acc_ref):
    i = pl.program_id(0)
    z = jnp.concatenate(
        [h_ref[c, q] + a_ref[c, q] for c in range(NC) for q in range(NQ)],
        axis=1)
    z = jnp.maximum(
        jnp.dot(z, w1_ref[...], preferred_element_type=jnp.float32)
        + b1_ref[...], 0.0)
    z = jnp.maximum(
        jnp.dot(z, w2_ref[...], preferred_element_type=jnp.float32)
        + b2_ref[...], 0.0)
    z_ref[...] = z
    s = jnp.sum(z, axis=0, keepdims=True)
    sq = jnp.sum(z * z, axis=0, keepdims=True)

    @pl.when(i == 0)
    def _():
        acc_ref[0:1] = s
        acc_ref[1:2] = sq

    @pl.when(i > 0)
    def _():
        acc_ref[0:1] += s
        acc_ref[1:2] += sq

    sum_ref[...] = acc_ref[0:1]
    sq_ref[...] = acc_ref[1:2]


def _mlp2_body(z_ref, sum_ref, sq_ref, g_ref, be_ref, out_ref, split_ref):
    mean = sum_ref[...] / N
    var = sq_ref[...] / N - mean * mean
    t = jnp.tanh((z_ref[...] - mean) * lax.rsqrt(var + 1e-5) * g_ref[...]
                 + be_ref[...])
    out_ref[...] = t
    for c in range(NC):
        for q in range(NQ):
            split_ref[c, q] = t[:, (c * NQ + q) * QF:(c * NQ + q + 1) * QF]


def _mlp_call(h4, agg4, W1l, B1l, W2l, B2l, Gl, Bel):
    blk4 = pl.BlockSpec((NC, NQ, BR, QF), lambda i: (0, 0, i, 0))
    blkz = pl.BlockSpec((BR, D), lambda i: (i, 0))
    full = lambda shape: pl.BlockSpec(shape, lambda i: tuple(0 for _ in shape))
    z, s, sq = pl.pallas_call(
        _mlp1_body,
        grid=(NB,),
        in_specs=[blk4, blk4, full((D, D)), full((1, D)), full((D, D)),
                  full((1, D))],
        out_specs=(blkz, full((1, D)), full((1, D))),
        out_shape=(jax.ShapeDtypeStruct((N, D), jnp.float32),
                   jax.ShapeDtypeStruct((1, D), jnp.float32),
                   jax.ShapeDtypeStruct((1, D), jnp.float32)),
        scratch_shapes=[pltpu.VMEM((8, D), jnp.float32)],
    )(h4, agg4, W1l, B1l, W2l, B2l)
    return pl.pallas_call(
        _mlp2_body,
        grid=(NB,),
        in_specs=[blkz, full((1, D)), full((1, D)), full((1, D)),
                  full((1, D))],
        out_specs=(blkz, pl.BlockSpec((NC, NQ, BR, QF),
                                      lambda i: (0, 0, i, 0))),
        out_shape=(jax.ShapeDtypeStruct((N, D), jnp.float32),
                   jax.ShapeDtypeStruct((NC, NQ, NP, QF), jnp.float32)),
    )(z, s, sq, Gl, Bel)


def kernel(x, edge_index, W0, b0, W1, B1, W2, B2, G, Be):
    src2 = edge_index[0].reshape(NS, EPT)
    dst2 = edge_index[1].reshape(NS, EPT)

    h4 = _lin0_call(x, W0, b0.reshape(1, D))

    ws = (W1, B1.reshape(L, 1, D), W2, B2.reshape(L, 1, D),
          G.reshape(L, 1, D), Be.reshape(L, 1, D))

    def step(h4c, w):
        w1, b1, w2, b2, g, be = w
        agg4 = _sc_agg_call(h4c, src2, dst2)
        h_full, h4n = _mlp_call(h4c, agg4, w1, b1, w2, b2, g, be)
        return h4n, h_full

    _, hs = lax.scan(step, h4, ws)
    return (x, hs[0], hs[1], hs[2])


# speedup vs baseline: 1.1348x; 1.0126x over previous
import functools

import jax
import jax.numpy as jnp
from jax import lax
from jax.experimental import pallas as pl
from jax.experimental.pallas import tpu as pltpu
from jax.experimental.pallas import tpu_sc as plsc

N = 10000
E = 320000
D = 128
L = 3

NC = 2
NS = 16
NQ = 2
QF = D // (NC * NQ)

NP = 10240
BR = 2000
NB = N // BR
EPT = E // NS
RPT = NP // NS
ZR = 128


CB = 128
NFULL = EPT // CB
NBUF = 6
NQUAD = NFULL // NBUF
TAIL = EPT - NFULL * CB


def _sc_agg_call(h4, src2, dst2):
    mesh = plsc.VectorSubcoreMesh(
        core_axis_name="c", subcore_axis_name="s", num_cores=NC, num_subcores=NS
    )

    @functools.partial(
        pl.kernel,
        mesh=mesh,
        compiler_params=pltpu.CompilerParams(use_tc_tiling_on_sc=False),
        out_type=jax.ShapeDtypeStruct((NC, NQ, NP, QF), jnp.float32),
        scratch_types=[
            pltpu.VMEM((EPT,), jnp.int32),
            pltpu.VMEM((EPT,), jnp.int32),
            [pltpu.VMEM((CB,), jnp.int32) for _ in range(NBUF)],
            [pltpu.VMEM((CB,), jnp.int32) for _ in range(NBUF)],
            [pltpu.VMEM((CB, QF), jnp.float32) for _ in range(NBUF)],
            pltpu.VMEM((TAIL,), jnp.int32),
            pltpu.VMEM((TAIL,), jnp.int32),
            pltpu.VMEM((TAIL, QF), jnp.float32),
            pltpu.VMEM((RPT, QF), jnp.float32),
            pltpu.VMEM_SHARED((NP, QF), jnp.float32),
            [pltpu.SemaphoreType.DMA for _ in range(NBUF)],
            [pltpu.SemaphoreType.DMA for _ in range(NBUF)],
        ],
    )
    def k(h_hbm, src_hbm, dst_hbm, out_hbm, src_v, dst_v, srcb, dstb, rows,
          srct_v, dstt_v, rows_t, bounce, agg_sh, gsem, ssem):
        cid = lax.axis_index("c")
        sid = lax.axis_index("s")

        pltpu.sync_copy(src_hbm.at[sid], src_v)
        pltpu.sync_copy(dst_hbm.at[sid], dst_v)

        def zstore(i, _):
            r = i // (QF // 16)
            col = (i % (QF // 16)) * 16
            bounce[r, pl.ds(col, 16)] = jnp.zeros((16,), jnp.float32)
            return 0

        lax.fori_loop(0, RPT * (QF // 16), zstore, 0)

        for q in range(NQ):
            pltpu.sync_copy(bounce, agg_sh.at[pl.ds(sid * RPT, RPT)])

            plsc.subcore_barrier()

            base = (cid * NQ + q) * NP

            def prep_src(buf, j):
                for kk in range(CB // 16):
                    buf[pl.ds(kk * 16, 16)] = (
                        src_v[pl.ds(j * CB + kk * 16, 16)] + base)

            def prep_dst(buf, j):
                for kk in range(CB // 16):
                    buf[pl.ds(kk * 16, 16)] = dst_v[pl.ds(j * CB + kk * 16, 16)]

            for b in range(NBUF):
                prep_src(srcb[b], b)
                pltpu.async_copy(h_hbm.at[srcb[b]], rows[b], gsem[b])

            def quad(i, _):
                j0 = i * NBUF
                for b in range(NBUF):
                    pltpu.make_async_copy(
                        h_hbm.at[srcb[b]], rows[b], gsem[b]).wait()
                    pltpu.async_copy(
                        rows[b], agg_sh.at[dst_v.at[pl.ds((j0 + b) * CB, CB)]],
                        ssem[b], add=True)
                for b in range(NBUF):
                    @pl.when(j0 + b + NBUF < NFULL)
                    def _():
                        pltpu.make_async_copy(
                            rows[b], agg_sh.at[dst_v.at[pl.ds((j0 + b) * CB, CB)]],
                            ssem[b]).wait()
                        prep_src(srcb[b], j0 + b + NBUF)
                        pltpu.async_copy(h_hbm.at[srcb[b]], rows[b], gsem[b])
                return 0

            lax.fori_loop(0, NQUAD, quad, 0)

            for b in range(NBUF):
                pltpu.make_async_copy(
                    rows[b], agg_sh.at[dst_v.at[pl.ds(b * CB, CB)]],
                    ssem[b]).wait()

            for kk in range(TAIL // 16):
                srct_v[pl.ds(kk * 16, 16)] = (
                    src_v[pl.ds(NFULL * CB + kk * 16, 16)] + base)
                dstt_v[pl.ds(kk * 16, 16)] = (
                    dst_v[pl.ds(NFULL * CB + kk * 16, 16)])
            pltpu.async_copy(h_hbm.at[srct_v], rows_t, gsem[0]).wait()
            pltpu.sync_copy(rows_t, agg_sh.at[dstt_v], add=True)

            plsc.subcore_barrier()

            pltpu.sync_copy(agg_sh.at[pl.ds(sid * RPT, RPT)], bounce)
            pltpu.sync_copy(bounce, out_hbm.at[cid, q, pl.ds(sid * RPT, RPT)])

            if q + 1 < NQ:
                lax.fori_loop(0, RPT * (QF // 16), zstore, 0)
                plsc.subcore_barrier()

    return k(h4.reshape(NC * NQ * NP, QF), src2, dst2)


def _lin0_body(x_ref, w_ref, b_ref, out_ref):
    h = jnp.dot(x_ref[...], w_ref[...], preferred_element_type=jnp.float32) + b_ref[...]
    for c in range(NC):
        for q in range(NQ):
            out_ref[c, q] = h[:, (c * NQ + q) * QF:(c * NQ + q + 1) * QF]


def _lin0_call(x, W0, b0):
    full = lambda shape: pl.BlockSpec(shape, lambda i: tuple(0 for _ in shape))
    return pl.pallas_call(
        _lin0_body,
        grid=(NB,),
        in_specs=[pl.BlockSpec((BR, D), lambda i: (i, 0)), full((D, D)),
                  full((1, D))],
        out_specs=pl.BlockSpec((NC, NQ, BR, QF), lambda i: (0, 0, i, 0)),
        out_shape=jax.ShapeDtypeStruct((NC, NQ, NP, QF), jnp.float32),
    )(x, W0, b0)


def _mlp1_body(h_ref, a_ref, w1_ref, b1_ref, w2_ref, b2_ref,
               z_ref, sum_ref, sq_ref, acc_ref):
    i = pl.program_id(0)
    z = jnp.concatenate(
        [h_ref[c, q] + a_ref[c, q] for c in range(NC) for q in range(NQ)],
        axis=1)
    z = jnp.maximum(
        jnp.dot(z, w1_ref[...], preferred_element_type=jnp.float32)
        + b1_ref[...], 0.0)
    z = jnp.maximum(
        jnp.dot(z, w2_ref[...], preferred_element_type=jnp.float32)
        + b2_ref[...], 0.0)
    z_ref[...] = z
    s = jnp.sum(z, axis=0, keepdims=True)
    sq = jnp.sum(z * z, axis=0, keepdims=True)

    @pl.when(i == 0)
    def _():
        acc_ref[0:1] = s
        acc_ref[1:2] = sq

    @pl.when(i > 0)
    def _():
        acc_ref[0:1] += s
        acc_ref[1:2] += sq

    sum_ref[...] = acc_ref[0:1]
    sq_ref[...] = acc_ref[1:2]


def _mlp2_body(z_ref, sum_ref, sq_ref, g_ref, be_ref, out_ref, split_ref):
    mean = sum_ref[...] / N
    var = sq_ref[...] / N - mean * mean
    t = jnp.tanh((z_ref[...] - mean) * lax.rsqrt(var + 1e-5) * g_ref[...]
                 + be_ref[...])
    out_ref[...] = t
    for c in range(NC):
        for q in range(NQ):
            split_ref[c, q] = t[:, (c * NQ + q) * QF:(c * NQ + q + 1) * QF]


def _mlp_call(h4, agg4, W1l, B1l, W2l, B2l, Gl, Bel):
    blk4 = pl.BlockSpec((NC, NQ, BR, QF), lambda i: (0, 0, i, 0))
    blkz = pl.BlockSpec((BR, D), lambda i: (i, 0))
    full = lambda shape: pl.BlockSpec(shape, lambda i: tuple(0 for _ in shape))
    z, s, sq = pl.pallas_call(
        _mlp1_body,
        grid=(NB,),
        in_specs=[blk4, blk4, full((D, D)), full((1, D)), full((D, D)),
                  full((1, D))],
        out_specs=(blkz, full((1, D)), full((1, D))),
        out_shape=(jax.ShapeDtypeStruct((N, D), jnp.float32),
                   jax.ShapeDtypeStruct((1, D), jnp.float32),
                   jax.ShapeDtypeStruct((1, D), jnp.float32)),
        scratch_shapes=[pltpu.VMEM((8, D), jnp.float32)],
    )(h4, agg4, W1l, B1l, W2l, B2l)
    return pl.pallas_call(
        _mlp2_body,
        grid=(NB,),
        in_specs=[blkz, full((1, D)), full((1, D)), full((1, D)),
                  full((1, D))],
        out_specs=(blkz, pl.BlockSpec((NC, NQ, BR, QF),
                                      lambda i: (0, 0, i, 0))),
        out_shape=(jax.ShapeDtypeStruct((N, D), jnp.float32),
                   jax.ShapeDtypeStruct((NC, NQ, NP, QF), jnp.float32)),
    )(z, s, sq, Gl, Bel)


def kernel(x, edge_index, W0, b0, W1, B1, W2, B2, G, Be):
    src2 = edge_index[0].reshape(NS, EPT)
    dst2 = edge_index[1].reshape(NS, EPT)

    h4 = _lin0_call(x, W0, b0.reshape(1, D))

    ws = (W1, B1.reshape(L, 1, D), W2, B2.reshape(L, 1, D),
          G.reshape(L, 1, D), Be.reshape(L, 1, D))

    def step(h4c, w):
        w1, b1, w2, b2, g, be = w
        agg4 = _sc_agg_call(h4c, src2, dst2)
        h_full, h4n = _mlp_call(h4c, agg4, w1, b1, w2, b2, g, be)
        return h4n, h_full

    _, hs = lax.scan(step, h4, ws)
    return (x, hs[0], hs[1], hs[2])
